# Initial kernel scaffold; baseline (speedup 1.0000x reference)
import jax
import jax.numpy as jnp
from jax.experimental import pallas as pl


def kernel(x, y, t, table):
    raise NotImplementedError("write your pallas kernel here")


# speedup vs baseline: 2.5898x; 2.5898x over previous
import jax
import jax.numpy as jnp
from jax import lax
from jax.experimental import pallas as pl
from jax.experimental.pallas import tpu as pltpu
from jax.experimental.pallas import tpu_sc as plsc

_EMBED_DIM = 32
_CLIP_HI = 1.0 - 1e-06
_INV_BIN = 32.0

_NC = 2
_NS = 16
_NW = _NC * _NS
_CHUNK = 2048
_GSIZE = 128
_NG = _CHUNK // _GSIZE


def _sc_body(x_hbm, y_hbm, t_hbm, table_hbm, out_hbm, xv, yv, tv, idxv, rows, sem):
    n = out_hbm.shape[0]
    per_w = n // _NW
    wid = lax.axis_index("s") * _NC + lax.axis_index("c")
    w_base = wid * per_w

    def outer(g, carry):
        base = w_base + g * _CHUNK
        pltpu.sync_copy(x_hbm.at[pl.ds(base, _CHUNK)], xv)
        pltpu.sync_copy(y_hbm.at[pl.ds(base, _CHUNK)], yv)
        pltpu.sync_copy(t_hbm.at[pl.ds(base, _CHUNK)], tv)

        def inner(i, c):
            s = pl.ds(i * 16, 16)
            xs = xv[s]
            ys = yv[s]
            ts = tv[s]
            vx = (jnp.minimum(jnp.maximum(xs, 0.0), _CLIP_HI) * _INV_BIN).astype(jnp.int32)
            vy = (jnp.minimum(jnp.maximum(ys, 0.0), _CLIP_HI) * _INV_BIN).astype(jnp.int32)
            a = xs + ys * vx.astype(jnp.float32)
            ti = (ts * vx) * vy
            idxv[s] = (a + ti.astype(jnp.float32)).astype(jnp.int32)
            return c

        lax.fori_loop(0, _CHUNK // 16, inner, 0)

        copies = []
        for j in range(_NG):
            sl = pl.ds(j * _GSIZE, _GSIZE)
            copies.append(pltpu.async_copy(table_hbm.at[idxv.at[sl]], rows.at[sl], sem))
        for c in copies:
            c.wait()
        pltpu.sync_copy(rows, out_hbm.at[pl.ds(base, _CHUNK)])
        return carry

    lax.fori_loop(0, per_w // _CHUNK, outer, 0)


def kernel(x, y, t, table):
    n = x.size
    xf = x.reshape(n)
    yf = y.reshape(n)
    tf = t.reshape(n)
    mesh = plsc.VectorSubcoreMesh(core_axis_name="c", subcore_axis_name="s")
    k = pl.kernel(
        _sc_body,
        out_type=jax.ShapeDtypeStruct((n, _EMBED_DIM), jnp.float32),
        mesh=mesh,
        scratch_types=[
            pltpu.VMEM((_CHUNK,), jnp.float32),
            pltpu.VMEM((_CHUNK,), jnp.float32),
            pltpu.VMEM((_CHUNK,), jnp.int32),
            pltpu.VMEM((_CHUNK,), jnp.int32),
            pltpu.VMEM((_CHUNK, _EMBED_DIM), jnp.float32),
            pltpu.SemaphoreType.DMA,
        ],
        compiler_params=pltpu.CompilerParams(use_tc_tiling_on_sc=False),
    )
    out = k(xf, yf, tf, table)
    return out.reshape(x.shape[0], x.shape[1], _EMBED_DIM)
